# Initial kernel scaffold; baseline (speedup 1.0000x reference)
import jax
import jax.numpy as jnp
from jax.experimental import pallas as pl


def kernel(x, edge_index, W1, b1, W2, b2):
    raise NotImplementedError("write your pallas kernel here")


# speedup vs baseline: 11.4906x; 11.4906x over previous
import functools

import jax
import jax.numpy as jnp
from jax import lax
from jax.experimental import pallas as pl
from jax.experimental.pallas import tpu as pltpu
from jax.experimental.pallas import tpu_sc as plsc

N = 10000
E = 160000
NPAD = 10240
EPAD = 163840
DIN = 256
DHID = 256
DOUT = 64
NC = 2
NS = 16
CHUNK = 128
RPT = NPAD // NS
BLK = 1024

_MESH = dict(core_axis_name="c", subcore_axis_name="s")


def _deg_body(dst_hbm, deg_out, idx_v, ones_v, zrow_v, acc):
    c = lax.axis_index("c")
    s = lax.axis_index("s")

    def fill_ones(i, _):
        ones_v[pl.ds(i * 16, 16)] = jnp.full((16,), 1.0, jnp.float32)
        return 0

    lax.fori_loop(0, CHUNK // 16, fill_ones, 0)

    def fill_zero(i, _):
        zrow_v[pl.ds(i * 16, 16)] = jnp.zeros((16,), jnp.float32)
        return 0

    lax.fori_loop(0, RPT // 16, fill_zero, 0)
    pltpu.sync_copy(zrow_v, acc.at[pl.ds(s * RPT, RPT)])
    plsc.subcore_barrier()

    ept = EPAD // (NC * NS)
    base = c * (EPAD // NC) + s * ept

    def step(g, _):
        pltpu.sync_copy(dst_hbm.at[pl.ds(base + g * CHUNK, CHUNK)], idx_v.at[0])
        pltpu.sync_copy(ones_v, acc.at[idx_v.at[0]], add=True)
        return 0

    lax.fori_loop(0, ept // CHUNK, step, 0)
    plsc.subcore_barrier()
    pltpu.sync_copy(acc.at[pl.ds(s * RPT, RPT)],
                    deg_out.at[pl.ds(c * NPAD + s * RPT, RPT)])


_deg_kernel = pl.kernel(
    _deg_body,
    out_type=jax.ShapeDtypeStruct((NC * NPAD,), jnp.float32),
    mesh=plsc.VectorSubcoreMesh(**_MESH),
    scratch_types=[
        pltpu.MemorySpace.VMEM((1, CHUNK), jnp.int32),
        pltpu.MemorySpace.VMEM((CHUNK,), jnp.float32),
        pltpu.MemorySpace.VMEM((RPT,), jnp.float32),
        pltpu.MemorySpace.VMEM_SHARED((NPAD,), jnp.float32),
    ],
)


def _agg_body(src_stride, dst_stride, ept,
              g_hbm, src_hbm, dst_hbm, z_hbm, out_hbm,
              sidx, didx, rows, acc, sem):
    c = lax.axis_index("c")
    s = lax.axis_index("s")
    pltpu.sync_copy(z_hbm, acc.at[pl.ds(s * RPT, RPT)])
    plsc.subcore_barrier()

    sbase = c * src_stride + s * ept
    dbase = c * dst_stride + s * ept

    def step(g, _):
        pltpu.sync_copy(src_hbm.at[pl.ds(sbase + g * CHUNK, CHUNK)], sidx.at[0])
        pltpu.sync_copy(dst_hbm.at[pl.ds(dbase + g * CHUNK, CHUNK)], didx.at[0])
        pltpu.async_copy(g_hbm.at[sidx.at[0]], rows, sem).wait()
        pltpu.sync_copy(rows, acc.at[didx.at[0]], add=True)
        return 0

    lax.fori_loop(0, ept // CHUNK, step, 0)
    plsc.subcore_barrier()
    pltpu.sync_copy(acc.at[pl.ds(s * RPT, RPT)],
                    out_hbm.at[pl.ds(c * NPAD + s * RPT, RPT)])


def _make_agg(width, src_stride, dst_stride, ept):
    params = pltpu.CompilerParams(use_tc_tiling_on_sc=(width % 128 == 0))
    return pl.kernel(
        functools.partial(_agg_body, src_stride, dst_stride, ept),
        out_type=jax.ShapeDtypeStruct((NC * NPAD, width), jnp.float32),
        mesh=plsc.VectorSubcoreMesh(**_MESH),
        compiler_params=params,
        scratch_types=[
            pltpu.MemorySpace.VMEM((1, CHUNK), jnp.int32),
            pltpu.MemorySpace.VMEM((1, CHUNK), jnp.int32),
            pltpu.MemorySpace.VMEM((CHUNK, width), jnp.float32),
            pltpu.MemorySpace.VMEM_SHARED((NPAD, width), jnp.float32),
            pltpu.SemaphoreType.DMA,
        ],
    )


_agg_l1 = _make_agg(128, src_stride=EPAD, dst_stride=0, ept=EPAD // NS)
_agg_l2 = _make_agg(DOUT, src_stride=EPAD // NC, dst_stride=EPAD // NC,
                    ept=EPAD // (NC * NS))


def _dinv_block(deg_ref, i):
    d = deg_ref[0, pl.ds(i * BLK, BLK)] + deg_ref[1, pl.ds(i * BLK, BLK)] + 1.0
    return lax.rsqrt(d)


def _k2_body(x_ref, w_ref, deg_ref, out_ref):
    i = pl.program_id(0)
    dinv = _dinv_block(deg_ref, i)
    h = jnp.dot(x_ref[...], w_ref[...], preferred_element_type=jnp.float32)
    out_ref[...] = h * dinv[:, None]


def _k2_call(x_pad, W1, deg):
    return pl.pallas_call(
        _k2_body,
        grid=(NPAD // BLK, 2),
        in_specs=[
            pl.BlockSpec((BLK, DIN), lambda i, j: (i, 0)),
            pl.BlockSpec((DIN, 128), lambda i, j: (0, j)),
            pl.BlockSpec((2, NPAD), lambda i, j: (0, 0)),
        ],
        out_specs=pl.BlockSpec((BLK, 128), lambda i, j: (j * (NPAD // BLK) + i, 0)),
        out_shape=jax.ShapeDtypeStruct((NC * NPAD, 128), jnp.float32),
    )(x_pad, W1, deg)


def _k4_body(alo, ahi, glo, ghi, mlo, mhi, deg_ref, b1_ref, w2_ref, out_ref):
    i = pl.program_id(0)
    dinv = _dinv_block(deg_ref, i)[:, None]
    zlo = mlo[0] * (2.0 * jnp.maximum(
        dinv * (alo[0] + glo[0]) + b1_ref[0, pl.ds(0, 128)][None, :], 0.0))
    zhi = mhi[0] * (2.0 * jnp.maximum(
        dinv * (ahi[0] + ghi[0]) + b1_ref[0, pl.ds(128, 128)][None, :], 0.0))
    h2 = (jnp.dot(zlo, w2_ref[pl.ds(0, 128), :], preferred_element_type=jnp.float32)
          + jnp.dot(zhi, w2_ref[pl.ds(128, 128), :], preferred_element_type=jnp.float32))
    out_ref[...] = h2 * dinv


def _k4_call(agg1, g1, mask3, deg, b1, W2):
    half = lambda h: pl.BlockSpec((1, BLK, 128), lambda i, h=h: (h, i, 0))
    return pl.pallas_call(
        _k4_body,
        grid=(NPAD // BLK,),
        in_specs=[
            half(0), half(1),
            half(0), half(1),
            half(0), half(1),
            pl.BlockSpec((2, NPAD), lambda i: (0, 0)),
            pl.BlockSpec((1, DHID), lambda i: (0, 0)),
            pl.BlockSpec((DHID, DOUT), lambda i: (0, 0)),
        ],
        out_specs=pl.BlockSpec((BLK, DOUT), lambda i: (i, 0)),
        out_shape=jax.ShapeDtypeStruct((NPAD, DOUT), jnp.float32),
    )(agg1, agg1, g1, g1, mask3, mask3, deg, b1, W2)


def _k6_body(p_ref, g2_ref, deg_ref, b2_ref, out_ref):
    i = pl.program_id(0)
    dinv = _dinv_block(deg_ref, i)[:, None]
    out_ref[...] = dinv * (p_ref[0] + p_ref[1] + g2_ref[...]) + b2_ref[0][None, :]


def _k6_call(agg2, g2, deg, b2):
    return pl.pallas_call(
        _k6_body,
        grid=(NPAD // BLK,),
        in_specs=[
            pl.BlockSpec((2, BLK, DOUT), lambda i: (0, i, 0)),
            pl.BlockSpec((BLK, DOUT), lambda i: (i, 0)),
            pl.BlockSpec((2, NPAD), lambda i: (0, 0)),
            pl.BlockSpec((1, DOUT), lambda i: (0, 0)),
        ],
        out_specs=pl.BlockSpec((BLK, DOUT), lambda i: (i, 0)),
        out_shape=jax.ShapeDtypeStruct((N, DOUT), jnp.float32),
    )(agg2, g2, deg, b2)


def kernel(x, edge_index, W1, b1, W2, b2):
    src = edge_index[0].astype(jnp.int32)
    dst = edge_index[1].astype(jnp.int32)
    pad_idx = (jnp.arange(EPAD - E, dtype=jnp.int32) % 192) + (NPAD - 192)
    srcp = jnp.concatenate([src, pad_idx])
    dstp = jnp.concatenate([dst, pad_idx])
    src2 = jnp.concatenate([srcp, srcp + NPAD])

    x_pad = jnp.pad(x, ((0, NPAD - N), (0, 0)))
    mask = jax.random.bernoulli(jax.random.key(42), 0.5, (N, DHID))
    mask_pad = jnp.pad(mask.astype(jnp.float32), ((0, NPAD - N), (0, 0)))
    mask3 = jnp.stack([mask_pad[:, :128], mask_pad[:, 128:]])
    z128 = jnp.zeros((RPT, 128), jnp.float32)
    z64 = jnp.zeros((RPT, DOUT), jnp.float32)

    deg = _deg_kernel(dstp).reshape(2, NPAD)
    g1 = _k2_call(x_pad, W1, deg)
    agg1 = _agg_l1(g1, src2, dstp, z128)
    g2 = _k4_call(agg1.reshape(2, NPAD, 128), g1.reshape(2, NPAD, 128),
                  mask3, deg, b1.reshape(1, DHID), W2)
    agg2 = _agg_l2(g2, srcp, dstp, z64)
    return _k6_call(agg2.reshape(2, NPAD, DOUT), g2, deg, b2.reshape(1, DOUT))


# speedup vs baseline: 19.6181x; 1.7073x over previous
import functools

import jax
import jax.numpy as jnp
from jax import lax
from jax.experimental import pallas as pl
from jax.experimental.pallas import tpu as pltpu
from jax.experimental.pallas import tpu_sc as plsc

N = 10000
E = 160000
NPAD = 10240
EPAD = 163840
DIN = 256
DHID = 256
DOUT = 64
NC = 2
NS = 16
CHUNK = 128
RPT = NPAD // NS
BLK = 1024

_MESH = dict(core_axis_name="c", subcore_axis_name="s")


def _deg_body(dst_hbm, deg_out, idx_v, ones_v, zrow_v, acc):
    c = lax.axis_index("c")
    s = lax.axis_index("s")

    def fill_ones(i, _):
        ones_v[pl.ds(i * 16, 16)] = jnp.full((16,), 1.0, jnp.float32)
        return 0

    lax.fori_loop(0, CHUNK // 16, fill_ones, 0)

    def fill_zero(i, _):
        zrow_v[pl.ds(i * 16, 16)] = jnp.zeros((16,), jnp.float32)
        return 0

    lax.fori_loop(0, RPT // 16, fill_zero, 0)
    pltpu.sync_copy(zrow_v, acc.at[pl.ds(s * RPT, RPT)])
    plsc.subcore_barrier()

    ept = EPAD // (NC * NS)
    base = c * (EPAD // NC) + s * ept

    def step(g, _):
        pltpu.sync_copy(dst_hbm.at[pl.ds(base + g * CHUNK, CHUNK)], idx_v.at[0])
        pltpu.sync_copy(ones_v, acc.at[idx_v.at[0]], add=True)
        return 0

    lax.fori_loop(0, ept // CHUNK, step, 0)
    plsc.subcore_barrier()
    pltpu.sync_copy(acc.at[pl.ds(s * RPT, RPT)],
                    deg_out.at[pl.ds(c * NPAD + s * RPT, RPT)])


_deg_kernel = pl.kernel(
    _deg_body,
    out_type=jax.ShapeDtypeStruct((NC * NPAD,), jnp.float32),
    mesh=plsc.VectorSubcoreMesh(**_MESH),
    scratch_types=[
        pltpu.MemorySpace.VMEM((1, CHUNK), jnp.int32),
        pltpu.MemorySpace.VMEM((CHUNK,), jnp.float32),
        pltpu.MemorySpace.VMEM((RPT,), jnp.float32),
        pltpu.MemorySpace.VMEM_SHARED((NPAD,), jnp.float32),
    ],
)


NBUF = 2


def _agg_body(src_cstride, dst_cstride, cpt, n_passes,
              g_hbm, src_hbm, dst_hbm, z_hbm, out_hbm,
              sidx, didx, rows0, rows1, acc, sem0, sem1):
    c = lax.axis_index("c")
    s = lax.axis_index("s")
    ppt = cpt // n_passes
    pltpu.sync_copy(z_hbm, acc.at[pl.ds(s * RPT, RPT)])
    plsc.subcore_barrier()

    rows = (rows0, rows1)
    sems = (sem0, sem1)
    for p in range(n_passes):
        sbase = c * src_cstride + s * cpt + p * ppt
        dbase = c * dst_cstride + s * cpt + p * ppt
        pltpu.sync_copy(src_hbm.at[pl.ds(sbase, ppt)], sidx)
        pltpu.sync_copy(dst_hbm.at[pl.ds(dbase, ppt)], didx)
        for b in range(NBUF):
            pltpu.async_copy(g_hbm.at[sidx.at[b]], rows[b], sems[b])

        def macro(m, _):
            for b in range(NBUF):
                ch = m * NBUF + b
                pltpu.make_async_copy(
                    g_hbm.at[sidx.at[ch]], rows[b], sems[b]).wait()
                pltpu.sync_copy(rows[b], acc.at[didx.at[ch]], add=True)
                pltpu.async_copy(g_hbm.at[sidx.at[ch + NBUF]], rows[b], sems[b])
            return 0

        lax.fori_loop(0, ppt // NBUF - 1, macro, 0)
        for b in range(NBUF):
            ch = ppt - NBUF + b
            pltpu.make_async_copy(g_hbm.at[sidx.at[ch]], rows[b], sems[b]).wait()
            pltpu.sync_copy(rows[b], acc.at[didx.at[ch]], add=True)

    plsc.subcore_barrier()
    pltpu.sync_copy(acc.at[pl.ds(s * RPT, RPT)],
                    out_hbm.at[pl.ds(c * NPAD + s * RPT, RPT)])


def _make_agg(width, src_cstride, dst_cstride, cpt, n_passes):
    params = pltpu.CompilerParams(use_tc_tiling_on_sc=(width % 128 == 0))
    return pl.kernel(
        functools.partial(_agg_body, src_cstride, dst_cstride, cpt, n_passes),
        out_type=jax.ShapeDtypeStruct((NC * NPAD, width), jnp.float32),
        mesh=plsc.VectorSubcoreMesh(**_MESH),
        compiler_params=params,
        scratch_types=[
            pltpu.MemorySpace.VMEM((cpt // n_passes, CHUNK), jnp.int32),
            pltpu.MemorySpace.VMEM((cpt // n_passes, CHUNK), jnp.int32),
            pltpu.MemorySpace.VMEM((CHUNK, width), jnp.float32),
            pltpu.MemorySpace.VMEM((CHUNK, width), jnp.float32),
            pltpu.MemorySpace.VMEM_SHARED((NPAD, width), jnp.float32),
            pltpu.SemaphoreType.DMA,
            pltpu.SemaphoreType.DMA,
        ],
    )


_agg_l1 = _make_agg(128, src_cstride=EPAD // CHUNK, dst_cstride=0,
                    cpt=EPAD // NS // CHUNK, n_passes=2)
_agg_l2 = _make_agg(DOUT, src_cstride=EPAD // NC // CHUNK,
                    dst_cstride=EPAD // NC // CHUNK,
                    cpt=EPAD // (NC * NS) // CHUNK, n_passes=1)


def _dinv_block(deg_ref, i):
    d = deg_ref[0, pl.ds(i * BLK, BLK)] + deg_ref[1, pl.ds(i * BLK, BLK)] + 1.0
    return lax.rsqrt(d)


def _k2_body(x_ref, w_ref, deg_ref, out_ref):
    i = pl.program_id(0)
    dinv = _dinv_block(deg_ref, i)
    h = jnp.dot(x_ref[...], w_ref[...], preferred_element_type=jnp.float32)
    out_ref[...] = h * dinv[:, None]


def _k2_call(x_pad, W1, deg):
    return pl.pallas_call(
        _k2_body,
        grid=(NPAD // BLK, 2),
        in_specs=[
            pl.BlockSpec((BLK, DIN), lambda i, j: (i, 0)),
            pl.BlockSpec((DIN, 128), lambda i, j: (0, j)),
            pl.BlockSpec((2, NPAD), lambda i, j: (0, 0)),
        ],
        out_specs=pl.BlockSpec((BLK, 128), lambda i, j: (j * (NPAD // BLK) + i, 0)),
        out_shape=jax.ShapeDtypeStruct((NC * NPAD, 128), jnp.float32),
    )(x_pad, W1, deg)


def _k4_body(alo, ahi, glo, ghi, mlo, mhi, deg_ref, b1_ref, w2_ref, out_ref):
    i = pl.program_id(0)
    dinv = _dinv_block(deg_ref, i)[:, None]
    zlo = mlo[0] * (2.0 * jnp.maximum(
        dinv * (alo[0] + glo[0]) + b1_ref[0, pl.ds(0, 128)][None, :], 0.0))
    zhi = mhi[0] * (2.0 * jnp.maximum(
        dinv * (ahi[0] + ghi[0]) + b1_ref[0, pl.ds(128, 128)][None, :], 0.0))
    h2 = (jnp.dot(zlo, w2_ref[pl.ds(0, 128), :], preferred_element_type=jnp.float32)
          + jnp.dot(zhi, w2_ref[pl.ds(128, 128), :], preferred_element_type=jnp.float32))
    out_ref[...] = h2 * dinv


def _k4_call(agg1, g1, mask3, deg, b1, W2):
    half = lambda h: pl.BlockSpec((1, BLK, 128), lambda i, h=h: (h, i, 0))
    return pl.pallas_call(
        _k4_body,
        grid=(NPAD // BLK,),
        in_specs=[
            half(0), half(1),
            half(0), half(1),
            half(0), half(1),
            pl.BlockSpec((2, NPAD), lambda i: (0, 0)),
            pl.BlockSpec((1, DHID), lambda i: (0, 0)),
            pl.BlockSpec((DHID, DOUT), lambda i: (0, 0)),
        ],
        out_specs=pl.BlockSpec((BLK, DOUT), lambda i: (i, 0)),
        out_shape=jax.ShapeDtypeStruct((NPAD, DOUT), jnp.float32),
    )(agg1, agg1, g1, g1, mask3, mask3, deg, b1, W2)


def _k6_body(p_ref, g2_ref, deg_ref, b2_ref, out_ref):
    i = pl.program_id(0)
    dinv = _dinv_block(deg_ref, i)[:, None]
    out_ref[...] = dinv * (p_ref[0] + p_ref[1] + g2_ref[...]) + b2_ref[0][None, :]


def _k6_call(agg2, g2, deg, b2):
    return pl.pallas_call(
        _k6_body,
        grid=(NPAD // BLK,),
        in_specs=[
            pl.BlockSpec((2, BLK, DOUT), lambda i: (0, i, 0)),
            pl.BlockSpec((BLK, DOUT), lambda i: (i, 0)),
            pl.BlockSpec((2, NPAD), lambda i: (0, 0)),
            pl.BlockSpec((1, DOUT), lambda i: (0, 0)),
        ],
        out_specs=pl.BlockSpec((BLK, DOUT), lambda i: (i, 0)),
        out_shape=jax.ShapeDtypeStruct((N, DOUT), jnp.float32),
    )(agg2, g2, deg, b2)


def kernel(x, edge_index, W1, b1, W2, b2):
    src = edge_index[0].astype(jnp.int32)
    dst = edge_index[1].astype(jnp.int32)
    pad_idx = (jnp.arange(EPAD - E, dtype=jnp.int32) % 192) + (NPAD - 192)
    srcp = jnp.concatenate([src, pad_idx])
    dstp = jnp.concatenate([dst, pad_idx])
    src2 = jnp.concatenate([srcp, srcp + NPAD])

    x_pad = jnp.pad(x, ((0, NPAD - N), (0, 0)))
    mask = jax.random.bernoulli(jax.random.key(42), 0.5, (N, DHID))
    mask_pad = jnp.pad(mask.astype(jnp.float32), ((0, NPAD - N), (0, 0)))
    mask3 = jnp.stack([mask_pad[:, :128], mask_pad[:, 128:]])
    z128 = jnp.zeros((RPT, 128), jnp.float32)
    z64 = jnp.zeros((RPT, DOUT), jnp.float32)

    srcp2 = srcp.reshape(-1, CHUNK)
    dstp2 = dstp.reshape(-1, CHUNK)
    src22 = src2.reshape(-1, CHUNK)

    deg = _deg_kernel(dstp).reshape(2, NPAD)
    g1 = _k2_call(x_pad, W1, deg)
    agg1 = _agg_l1(g1, src22, dstp2, z128)
    g2 = _k4_call(agg1.reshape(2, NPAD, 128), g1.reshape(2, NPAD, 128),
                  mask3, deg, b1.reshape(1, DHID), W2)
    agg2 = _agg_l2(g2, srcp2, dstp2, z64)
    return _k6_call(agg2.reshape(2, NPAD, DOUT), g2, deg, b2.reshape(1, DOUT))


# speedup vs baseline: 21.6810x; 1.1052x over previous
import functools

import jax
import jax.numpy as jnp
from jax import lax
from jax.experimental import pallas as pl
from jax.experimental.pallas import tpu as pltpu
from jax.experimental.pallas import tpu_sc as plsc

N = 10000
E = 160000
NPAD = 10240
EPAD = 163840
DIN = 256
DHID = 256
DOUT = 64
NC = 2
NS = 16
CHUNK = 128
RPT = NPAD // NS
BLK = 1024

_MESH = dict(core_axis_name="c", subcore_axis_name="s")


_DEG_CPT = EPAD // (NC * NS) // CHUNK


def _deg_body(dst_hbm, deg_out, didx, ones_v, zrow_v, acc, sem):
    c = lax.axis_index("c")
    s = lax.axis_index("s")

    def fill_ones(i, _):
        ones_v[pl.ds(i * 16, 16)] = jnp.full((16,), 1.0, jnp.float32)
        return 0

    lax.fori_loop(0, CHUNK // 16, fill_ones, 0)

    def fill_zero(i, _):
        zrow_v[pl.ds(i * 16, 16)] = jnp.zeros((16,), jnp.float32)
        return 0

    lax.fori_loop(0, RPT // 16, fill_zero, 0)
    pltpu.sync_copy(
        dst_hbm.at[pl.ds(c * (NC * _DEG_CPT * NS // 2) + s * _DEG_CPT, _DEG_CPT)],
        didx)
    pltpu.sync_copy(zrow_v, acc.at[pl.ds(s * RPT, RPT)])
    plsc.subcore_barrier()

    def issue(g, _):
        pltpu.async_copy(ones_v, acc.at[didx.at[g]], sem, add=True)
        return 0

    lax.fori_loop(0, _DEG_CPT, issue, 0)

    def drain(g, _):
        pltpu.make_async_copy(ones_v, acc.at[didx.at[0]], sem).wait()
        return 0

    lax.fori_loop(0, _DEG_CPT, drain, 0)
    plsc.subcore_barrier()
    pltpu.sync_copy(acc.at[pl.ds(s * RPT, RPT)],
                    deg_out.at[pl.ds(c * NPAD + s * RPT, RPT)])


_deg_kernel = pl.kernel(
    _deg_body,
    out_type=jax.ShapeDtypeStruct((NC * NPAD,), jnp.float32),
    mesh=plsc.VectorSubcoreMesh(**_MESH),
    scratch_types=[
        pltpu.MemorySpace.VMEM((_DEG_CPT, CHUNK), jnp.int32),
        pltpu.MemorySpace.VMEM((CHUNK,), jnp.float32),
        pltpu.MemorySpace.VMEM((RPT,), jnp.float32),
        pltpu.MemorySpace.VMEM_SHARED((NPAD,), jnp.float32),
        pltpu.SemaphoreType.DMA,
    ],
)


def _agg_body(src_cstride, dst_cstride, cpt, n_passes, nbuf,
              g_hbm, src_hbm, dst_hbm, z_hbm, out_hbm,
              sidx, didx, acc, *bufs):
    c = lax.axis_index("c")
    s = lax.axis_index("s")
    ppt = cpt // n_passes
    rows = bufs[:nbuf]
    sems = bufs[nbuf:]
    pltpu.sync_copy(z_hbm, acc.at[pl.ds(s * RPT, RPT)])
    plsc.subcore_barrier()

    for p in range(n_passes):
        sbase = c * src_cstride + s * cpt + p * ppt
        dbase = c * dst_cstride + s * cpt + p * ppt
        pltpu.sync_copy(src_hbm.at[pl.ds(sbase, ppt)], sidx)
        pltpu.sync_copy(dst_hbm.at[pl.ds(dbase, ppt)], didx)
        for b in range(nbuf):
            pltpu.async_copy(g_hbm.at[sidx.at[b]], rows[b], sems[b])

        def macro(m, _):
            for b in range(nbuf):
                ch = m * nbuf + b
                pltpu.make_async_copy(
                    g_hbm.at[sidx.at[ch]], rows[b], sems[b]).wait()
                pltpu.sync_copy(rows[b], acc.at[didx.at[ch]], add=True)
                pltpu.async_copy(g_hbm.at[sidx.at[ch + nbuf]], rows[b], sems[b])
            return 0

        lax.fori_loop(0, ppt // nbuf - 1, macro, 0)
        for b in range(nbuf):
            ch = ppt - nbuf + b
            pltpu.make_async_copy(g_hbm.at[sidx.at[ch]], rows[b], sems[b]).wait()
            pltpu.sync_copy(rows[b], acc.at[didx.at[ch]], add=True)

    plsc.subcore_barrier()
    pltpu.sync_copy(acc.at[pl.ds(s * RPT, RPT)],
                    out_hbm.at[pl.ds(c * NPAD + s * RPT, RPT)])


def _make_agg(width, src_cstride, dst_cstride, cpt, n_passes, nbuf):
    params = pltpu.CompilerParams(use_tc_tiling_on_sc=(width % 128 == 0))
    return pl.kernel(
        functools.partial(_agg_body, src_cstride, dst_cstride, cpt, n_passes,
                          nbuf),
        out_type=jax.ShapeDtypeStruct((NC * NPAD, width), jnp.float32),
        mesh=plsc.VectorSubcoreMesh(**_MESH),
        compiler_params=params,
        scratch_types=(
            [pltpu.MemorySpace.VMEM((cpt // n_passes, CHUNK), jnp.int32),
             pltpu.MemorySpace.VMEM((cpt // n_passes, CHUNK), jnp.int32),
             pltpu.MemorySpace.VMEM_SHARED((NPAD, width), jnp.float32)]
            + [pltpu.MemorySpace.VMEM((CHUNK, width), jnp.float32)
               for _ in range(nbuf)]
            + [pltpu.SemaphoreType.DMA for _ in range(nbuf)]
        ),
    )


_agg_l1 = _make_agg(128, src_cstride=EPAD // CHUNK, dst_cstride=0,
                    cpt=EPAD // NS // CHUNK, n_passes=2, nbuf=2)
_agg_l2 = _make_agg(DOUT, src_cstride=EPAD // NC // CHUNK,
                    dst_cstride=EPAD // NC // CHUNK,
                    cpt=EPAD // (NC * NS) // CHUNK, n_passes=1, nbuf=4)


def _dinv_block(deg_ref, i):
    d = deg_ref[0, pl.ds(i * BLK, BLK)] + deg_ref[1, pl.ds(i * BLK, BLK)] + 1.0
    return lax.rsqrt(d)


def _k2_body(x_ref, w_ref, deg_ref, out_ref):
    i = pl.program_id(0)
    dinv = _dinv_block(deg_ref, i)
    h = jnp.dot(x_ref[...], w_ref[...], preferred_element_type=jnp.float32)
    out_ref[...] = h * dinv[:, None]


def _k2_call(x_pad, W1, deg):
    return pl.pallas_call(
        _k2_body,
        grid=(NPAD // BLK, 2),
        in_specs=[
            pl.BlockSpec((BLK, DIN), lambda i, j: (i, 0)),
            pl.BlockSpec((DIN, 128), lambda i, j: (0, j)),
            pl.BlockSpec((2, NPAD), lambda i, j: (0, 0)),
        ],
        out_specs=pl.BlockSpec((BLK, 128), lambda i, j: (j * (NPAD // BLK) + i, 0)),
        out_shape=jax.ShapeDtypeStruct((NC * NPAD, 128), jnp.float32),
    )(x_pad, W1, deg)


def _k4_body(alo, ahi, glo, ghi, mlo, mhi, deg_ref, b1_ref, w2_ref, out_ref):
    i = pl.program_id(0)
    dinv = _dinv_block(deg_ref, i)[:, None]
    zlo = mlo[0] * (2.0 * jnp.maximum(
        dinv * (alo[0] + glo[0]) + b1_ref[0, pl.ds(0, 128)][None, :], 0.0))
    zhi = mhi[0] * (2.0 * jnp.maximum(
        dinv * (ahi[0] + ghi[0]) + b1_ref[0, pl.ds(128, 128)][None, :], 0.0))
    h2 = (jnp.dot(zlo, w2_ref[pl.ds(0, 128), :], preferred_element_type=jnp.float32)
          + jnp.dot(zhi, w2_ref[pl.ds(128, 128), :], preferred_element_type=jnp.float32))
    out_ref[...] = h2 * dinv


def _k4_call(agg1, g1, mask3, deg, b1, W2):
    half = lambda h: pl.BlockSpec((1, BLK, 128), lambda i, h=h: (h, i, 0))
    return pl.pallas_call(
        _k4_body,
        grid=(NPAD // BLK,),
        in_specs=[
            half(0), half(1),
            half(0), half(1),
            half(0), half(1),
            pl.BlockSpec((2, NPAD), lambda i: (0, 0)),
            pl.BlockSpec((1, DHID), lambda i: (0, 0)),
            pl.BlockSpec((DHID, DOUT), lambda i: (0, 0)),
        ],
        out_specs=pl.BlockSpec((BLK, DOUT), lambda i: (i, 0)),
        out_shape=jax.ShapeDtypeStruct((NPAD, DOUT), jnp.float32),
    )(agg1, agg1, g1, g1, mask3, mask3, deg, b1, W2)


def _k6_body(p0_ref, p1_ref, g2_ref, deg_ref, b2_ref, out_ref):
    i = pl.program_id(0)
    dinv = _dinv_block(deg_ref, i)[:, None]
    out_ref[...] = (dinv * (p0_ref[...] + p1_ref[...] + g2_ref[...])
                    + b2_ref[0][None, :])


def _k6_call(agg2, g2, deg, b2):
    nb = NPAD // BLK
    return pl.pallas_call(
        _k6_body,
        grid=(nb,),
        in_specs=[
            pl.BlockSpec((BLK, DOUT), lambda i: (i, 0)),
            pl.BlockSpec((BLK, DOUT), lambda i: (nb + i, 0)),
            pl.BlockSpec((BLK, DOUT), lambda i: (i, 0)),
            pl.BlockSpec((2, NPAD), lambda i: (0, 0)),
            pl.BlockSpec((1, DOUT), lambda i: (0, 0)),
        ],
        out_specs=pl.BlockSpec((BLK, DOUT), lambda i: (i, 0)),
        out_shape=jax.ShapeDtypeStruct((N, DOUT), jnp.float32),
    )(agg2, agg2, g2, deg, b2)


def kernel(x, edge_index, W1, b1, W2, b2):
    src = edge_index[0].astype(jnp.int32)
    dst = edge_index[1].astype(jnp.int32)
    pad_idx = (jnp.arange(EPAD - E, dtype=jnp.int32) % 192) + (NPAD - 192)
    srcp = jnp.concatenate([src, pad_idx])
    dstp = jnp.concatenate([dst, pad_idx])
    src2 = jnp.concatenate([srcp, srcp + NPAD])

    x_pad = jnp.pad(x, ((0, NPAD - N), (0, 0)))
    mask = jax.random.bernoulli(jax.random.key(42), 0.5, (N, DHID))
    mask_pad = jnp.pad(mask.astype(jnp.float32), ((0, NPAD - N), (0, 0)))
    mask3 = jnp.stack([mask_pad[:, :128], mask_pad[:, 128:]])
    z128 = jnp.zeros((RPT, 128), jnp.float32)
    z64 = jnp.zeros((RPT, DOUT), jnp.float32)

    srcp2 = srcp.reshape(-1, CHUNK)
    dstp2 = dstp.reshape(-1, CHUNK)
    src22 = src2.reshape(-1, CHUNK)

    deg = _deg_kernel(dstp2).reshape(2, NPAD)
    g1 = _k2_call(x_pad, W1, deg)
    agg1 = _agg_l1(g1, src22, dstp2, z128)
    g2 = _k4_call(agg1.reshape(2, NPAD, 128), g1.reshape(2, NPAD, 128),
                  mask3, deg, b1.reshape(1, DHID), W2)
    agg2 = _agg_l2(g2, srcp2, dstp2, z64)
    return _k6_call(agg2, g2, deg, b2.reshape(1, DOUT))


# speedup vs baseline: 21.9581x; 1.0128x over previous
import functools

import jax
import jax.numpy as jnp
from jax import lax
from jax.experimental import pallas as pl
from jax.experimental.pallas import tpu as pltpu
from jax.experimental.pallas import tpu_sc as plsc

N = 10000
E = 160000
NPAD = 10240
EPAD = 163840
DIN = 256
DHID = 256
DOUT = 64
NC = 2
NS = 16
CHUNK = 128
RPT = NPAD // NS
BLK = 1024

_MESH = dict(core_axis_name="c", subcore_axis_name="s")


_DEG_CPT = EPAD // (NC * NS) // CHUNK


def _deg_body(dst_hbm, deg_out, didx, ones_v, zrow_v, acc, sem):
    c = lax.axis_index("c")
    s = lax.axis_index("s")

    def fill_ones(i, _):
        ones_v[pl.ds(i * 16, 16)] = jnp.full((16,), 1.0, jnp.float32)
        return 0

    lax.fori_loop(0, CHUNK // 16, fill_ones, 0)

    def fill_zero(i, _):
        zrow_v[pl.ds(i * 16, 16)] = jnp.zeros((16,), jnp.float32)
        return 0

    lax.fori_loop(0, RPT // 16, fill_zero, 0)
    pltpu.sync_copy(
        dst_hbm.at[pl.ds(c * (NC * _DEG_CPT * NS // 2) + s * _DEG_CPT, _DEG_CPT)],
        didx)
    pltpu.sync_copy(zrow_v, acc.at[pl.ds(s * RPT, RPT)])
    plsc.subcore_barrier()

    def issue(g, _):
        pltpu.async_copy(ones_v, acc.at[didx.at[g]], sem, add=True)
        return 0

    lax.fori_loop(0, _DEG_CPT, issue, 0)

    def drain(g, _):
        pltpu.make_async_copy(ones_v, acc.at[didx.at[0]], sem).wait()
        return 0

    lax.fori_loop(0, _DEG_CPT, drain, 0)
    plsc.subcore_barrier()
    pltpu.sync_copy(acc.at[pl.ds(s * RPT, RPT)],
                    deg_out.at[pl.ds(c * NPAD + s * RPT, RPT)])


_deg_kernel = pl.kernel(
    _deg_body,
    out_type=jax.ShapeDtypeStruct((NC * NPAD,), jnp.float32),
    mesh=plsc.VectorSubcoreMesh(**_MESH),
    scratch_types=[
        pltpu.MemorySpace.VMEM((_DEG_CPT, CHUNK), jnp.int32),
        pltpu.MemorySpace.VMEM((CHUNK,), jnp.float32),
        pltpu.MemorySpace.VMEM((RPT,), jnp.float32),
        pltpu.MemorySpace.VMEM_SHARED((NPAD,), jnp.float32),
        pltpu.SemaphoreType.DMA,
    ],
)


def _agg_body(src_cstride, dst_cstride, cpt, n_passes, nbuf,
              g_hbm, src_hbm, dst_hbm, z_hbm, out_hbm,
              sidx, didx, acc, *bufs):
    c = lax.axis_index("c")
    s = lax.axis_index("s")
    ppt = cpt // n_passes
    rows = bufs[:nbuf]
    sems = bufs[nbuf:]
    pltpu.sync_copy(z_hbm, acc.at[pl.ds(s * RPT, RPT)])
    plsc.subcore_barrier()

    for p in range(n_passes):
        sbase = c * src_cstride + s * cpt + p * ppt
        dbase = c * dst_cstride + s * cpt + p * ppt
        pltpu.sync_copy(src_hbm.at[pl.ds(sbase, ppt)], sidx)
        pltpu.sync_copy(dst_hbm.at[pl.ds(dbase, ppt)], didx)
        for b in range(nbuf):
            pltpu.async_copy(g_hbm.at[sidx.at[b]], rows[b], sems[b])

        def macro(m, _):
            for b in range(nbuf):
                ch = m * nbuf + b
                pltpu.make_async_copy(
                    g_hbm.at[sidx.at[ch]], rows[b], sems[b]).wait()
                pltpu.sync_copy(rows[b], acc.at[didx.at[ch]], add=True)
                pltpu.async_copy(g_hbm.at[sidx.at[ch + nbuf]], rows[b], sems[b])
            return 0

        lax.fori_loop(0, ppt // nbuf - 1, macro, 0)
        for b in range(nbuf):
            ch = ppt - nbuf + b
            pltpu.make_async_copy(g_hbm.at[sidx.at[ch]], rows[b], sems[b]).wait()
            pltpu.sync_copy(rows[b], acc.at[didx.at[ch]], add=True)

    plsc.subcore_barrier()
    pltpu.sync_copy(acc.at[pl.ds(s * RPT, RPT)],
                    out_hbm.at[pl.ds(c * NPAD + s * RPT, RPT)])


def _make_agg(width, src_cstride, dst_cstride, cpt, n_passes, nbuf):
    params = pltpu.CompilerParams(use_tc_tiling_on_sc=(width % 128 == 0))
    return pl.kernel(
        functools.partial(_agg_body, src_cstride, dst_cstride, cpt, n_passes,
                          nbuf),
        out_type=jax.ShapeDtypeStruct((NC * NPAD, width), jnp.float32),
        mesh=plsc.VectorSubcoreMesh(**_MESH),
        compiler_params=params,
        scratch_types=(
            [pltpu.MemorySpace.VMEM((cpt // n_passes, CHUNK), jnp.int32),
             pltpu.MemorySpace.VMEM((cpt // n_passes, CHUNK), jnp.int32),
             pltpu.MemorySpace.VMEM_SHARED((NPAD, width), jnp.float32)]
            + [pltpu.MemorySpace.VMEM((CHUNK, width), jnp.float32)
               for _ in range(nbuf)]
            + [pltpu.SemaphoreType.DMA for _ in range(nbuf)]
        ),
    )


_agg_l1 = _make_agg(128, src_cstride=EPAD // CHUNK, dst_cstride=0,
                    cpt=EPAD // NS // CHUNK, n_passes=2, nbuf=2)
_agg_l2 = _make_agg(DOUT, src_cstride=EPAD // NC // CHUNK,
                    dst_cstride=EPAD // NC // CHUNK,
                    cpt=EPAD // (NC * NS) // CHUNK, n_passes=1, nbuf=4)


def _dinv_block(deg_ref, i):
    d = deg_ref[0, pl.ds(i * BLK, BLK)] + deg_ref[1, pl.ds(i * BLK, BLK)] + 1.0
    return lax.rsqrt(d)


def _k2_body(x_ref, w_ref, deg_ref, out_ref):
    i = pl.program_id(0)
    dinv = _dinv_block(deg_ref, i)
    h = jnp.dot(x_ref[...], w_ref[...], preferred_element_type=jnp.float32)
    out_ref[...] = h * dinv[:, None]


def _k2_call(x, W1, deg):
    return pl.pallas_call(
        _k2_body,
        grid=(NPAD // BLK, 2),
        in_specs=[
            pl.BlockSpec((BLK, DIN), lambda i, j: (i, 0)),
            pl.BlockSpec((DIN, 128), lambda i, j: (0, j)),
            pl.BlockSpec((2, NPAD), lambda i, j: (0, 0)),
        ],
        out_specs=pl.BlockSpec((BLK, 128), lambda i, j: (j * (NPAD // BLK) + i, 0)),
        out_shape=jax.ShapeDtypeStruct((NC * NPAD, 128), jnp.float32),
    )(x, W1, deg)


def _k4_body(alo, ahi, glo, ghi, mlo, mhi, deg_ref, b1_ref, w2_ref, out_ref):
    i = pl.program_id(0)
    dinv = _dinv_block(deg_ref, i)[:, None]
    zlo = mlo[0] * (2.0 * jnp.maximum(
        dinv * (alo[0] + glo[0]) + b1_ref[0, pl.ds(0, 128)][None, :], 0.0))
    zhi = mhi[0] * (2.0 * jnp.maximum(
        dinv * (ahi[0] + ghi[0]) + b1_ref[0, pl.ds(128, 128)][None, :], 0.0))
    h2 = (jnp.dot(zlo, w2_ref[pl.ds(0, 128), :], preferred_element_type=jnp.float32)
          + jnp.dot(zhi, w2_ref[pl.ds(128, 128), :], preferred_element_type=jnp.float32))
    out_ref[...] = h2 * dinv


def _k4_call(agg1, g1, mask3, deg, b1, W2):
    half = lambda h: pl.BlockSpec((1, BLK, 128), lambda i, h=h: (h, i, 0))
    return pl.pallas_call(
        _k4_body,
        grid=(NPAD // BLK,),
        in_specs=[
            half(0), half(1),
            half(0), half(1),
            half(0), half(1),
            pl.BlockSpec((2, NPAD), lambda i: (0, 0)),
            pl.BlockSpec((1, DHID), lambda i: (0, 0)),
            pl.BlockSpec((DHID, DOUT), lambda i: (0, 0)),
        ],
        out_specs=pl.BlockSpec((BLK, DOUT), lambda i: (i, 0)),
        out_shape=jax.ShapeDtypeStruct((NPAD, DOUT), jnp.float32),
    )(agg1, agg1, g1, g1, mask3, mask3, deg, b1, W2)


def _k6_body(p0_ref, p1_ref, g2_ref, deg_ref, b2_ref, out_ref):
    i = pl.program_id(0)
    dinv = _dinv_block(deg_ref, i)[:, None]
    out_ref[...] = (dinv * (p0_ref[...] + p1_ref[...] + g2_ref[...])
                    + b2_ref[0][None, :])


def _k6_call(agg2, g2, deg, b2):
    nb = NPAD // BLK
    return pl.pallas_call(
        _k6_body,
        grid=(nb,),
        in_specs=[
            pl.BlockSpec((BLK, DOUT), lambda i: (i, 0)),
            pl.BlockSpec((BLK, DOUT), lambda i: (nb + i, 0)),
            pl.BlockSpec((BLK, DOUT), lambda i: (i, 0)),
            pl.BlockSpec((2, NPAD), lambda i: (0, 0)),
            pl.BlockSpec((1, DOUT), lambda i: (0, 0)),
        ],
        out_specs=pl.BlockSpec((BLK, DOUT), lambda i: (i, 0)),
        out_shape=jax.ShapeDtypeStruct((N, DOUT), jnp.float32),
    )(agg2, agg2, g2, deg, b2)


def kernel(x, edge_index, W1, b1, W2, b2):
    src = edge_index[0].astype(jnp.int32)
    dst = edge_index[1].astype(jnp.int32)
    pad_idx = (jnp.arange(EPAD - E, dtype=jnp.int32) % 192) + (NPAD - 192)
    srcp = jnp.concatenate([src, pad_idx])
    dstp = jnp.concatenate([dst, pad_idx])
    src2 = jnp.concatenate([srcp, srcp + NPAD])

    mask = jax.random.bernoulli(jax.random.key(42), 0.5, (N, DHID))
    mask_pad = jnp.pad(mask.astype(jnp.float32), ((0, NPAD - N), (0, 0)))
    mask3 = jnp.stack([mask_pad[:, :128], mask_pad[:, 128:]])
    z128 = jnp.zeros((RPT, 128), jnp.float32)
    z64 = jnp.zeros((RPT, DOUT), jnp.float32)

    srcp2 = srcp.reshape(-1, CHUNK)
    dstp2 = dstp.reshape(-1, CHUNK)
    src22 = src2.reshape(-1, CHUNK)

    deg = _deg_kernel(dstp2).reshape(2, NPAD)
    g1 = _k2_call(x, W1, deg)
    agg1 = _agg_l1(g1, src22, dstp2, z128)
    g2 = _k4_call(agg1.reshape(2, NPAD, 128), g1.reshape(2, NPAD, 128),
                  mask3, deg, b1.reshape(1, DHID), W2)
    agg2 = _agg_l2(g2, srcp2, dstp2, z64)
    return _k6_call(agg2, g2, deg, b2.reshape(1, DOUT))


# speedup vs baseline: 22.5417x; 1.0266x over previous
import functools

import jax
import jax.numpy as jnp
from jax import lax
from jax.experimental import pallas as pl
from jax.experimental.pallas import tpu as pltpu
from jax.experimental.pallas import tpu_sc as plsc

N = 10000
E = 160000
NPAD = 10240
EPAD = 163840
DIN = 256
DHID = 256
DOUT = 64
NC = 2
NS = 16
CHUNK = 128
RPT = NPAD // NS
BLK = 1024

_MESH = dict(core_axis_name="c", subcore_axis_name="s")


_DEG_CPT = EPAD // (NC * NS) // CHUNK


def _deg_body(dst_hbm, deg_out, didx, ones_v, zrow_v, acc, sem):
    c = lax.axis_index("c")
    s = lax.axis_index("s")

    def fill_ones(i, _):
        ones_v[pl.ds(i * 16, 16)] = jnp.full((16,), 1.0, jnp.float32)
        return 0

    lax.fori_loop(0, CHUNK // 16, fill_ones, 0)

    def fill_zero(i, _):
        zrow_v[pl.ds(i * 16, 16)] = jnp.zeros((16,), jnp.float32)
        return 0

    lax.fori_loop(0, RPT // 16, fill_zero, 0)
    pltpu.sync_copy(
        dst_hbm.at[pl.ds(c * (NC * _DEG_CPT * NS // 2) + s * _DEG_CPT, _DEG_CPT)],
        didx)
    pltpu.sync_copy(zrow_v, acc.at[pl.ds(s * RPT, RPT)])
    plsc.subcore_barrier()

    def issue(g, _):
        pltpu.async_copy(ones_v, acc.at[didx.at[g]], sem, add=True)
        return 0

    lax.fori_loop(0, _DEG_CPT, issue, 0)

    def drain(g, _):
        pltpu.make_async_copy(ones_v, acc.at[didx.at[0]], sem).wait()
        return 0

    lax.fori_loop(0, _DEG_CPT, drain, 0)
    plsc.subcore_barrier()
    pltpu.sync_copy(acc.at[pl.ds(s * RPT, RPT)],
                    deg_out.at[pl.ds(c * NPAD + s * RPT, RPT)])


_deg_kernel = pl.kernel(
    _deg_body,
    out_type=jax.ShapeDtypeStruct((NC * NPAD,), jnp.float32),
    mesh=plsc.VectorSubcoreMesh(**_MESH),
    scratch_types=[
        pltpu.MemorySpace.VMEM((_DEG_CPT, CHUNK), jnp.int32),
        pltpu.MemorySpace.VMEM((CHUNK,), jnp.float32),
        pltpu.MemorySpace.VMEM((RPT,), jnp.float32),
        pltpu.MemorySpace.VMEM_SHARED((NPAD,), jnp.float32),
        pltpu.SemaphoreType.DMA,
    ],
)


def _agg_body(src_cstride, dst_cstride, cpt, n_passes, nbuf,
              g_hbm, src_hbm, dst_hbm, z_hbm, out_hbm,
              sidx, didx, acc, *bufs):
    c = lax.axis_index("c")
    s = lax.axis_index("s")
    ppt = cpt // n_passes
    rows = bufs[:nbuf]
    sems = bufs[nbuf:]
    pltpu.sync_copy(z_hbm, acc.at[pl.ds(s * RPT, RPT)])
    plsc.subcore_barrier()

    for p in range(n_passes):
        sbase = c * src_cstride + s * cpt + p * ppt
        dbase = c * dst_cstride + s * cpt + p * ppt
        pltpu.sync_copy(src_hbm.at[pl.ds(sbase, ppt)], sidx)
        pltpu.sync_copy(dst_hbm.at[pl.ds(dbase, ppt)], didx)
        for b in range(nbuf):
            pltpu.async_copy(g_hbm.at[sidx.at[b]], rows[b], sems[b])

        def macro(m, _):
            for b in range(nbuf):
                ch = m * nbuf + b
                pltpu.make_async_copy(
                    g_hbm.at[sidx.at[ch]], rows[b], sems[b]).wait()
                pltpu.sync_copy(rows[b], acc.at[didx.at[ch]], add=True)
                pltpu.async_copy(g_hbm.at[sidx.at[ch + nbuf]], rows[b], sems[b])
            return 0

        lax.fori_loop(0, ppt // nbuf - 1, macro, 0)
        for b in range(nbuf):
            ch = ppt - nbuf + b
            pltpu.make_async_copy(g_hbm.at[sidx.at[ch]], rows[b], sems[b]).wait()
            pltpu.sync_copy(rows[b], acc.at[didx.at[ch]], add=True)

    plsc.subcore_barrier()
    pltpu.sync_copy(acc.at[pl.ds(s * RPT, RPT)],
                    out_hbm.at[pl.ds(c * NPAD + s * RPT, RPT)])


def _make_agg(width, src_cstride, dst_cstride, cpt, n_passes, nbuf):
    params = pltpu.CompilerParams(use_tc_tiling_on_sc=(width % 128 == 0))
    return pl.kernel(
        functools.partial(_agg_body, src_cstride, dst_cstride, cpt, n_passes,
                          nbuf),
        out_type=jax.ShapeDtypeStruct((NC * NPAD, width), jnp.float32),
        mesh=plsc.VectorSubcoreMesh(**_MESH),
        compiler_params=params,
        scratch_types=(
            [pltpu.MemorySpace.VMEM((cpt // n_passes, CHUNK), jnp.int32),
             pltpu.MemorySpace.VMEM((cpt // n_passes, CHUNK), jnp.int32),
             pltpu.MemorySpace.VMEM_SHARED((NPAD, width), jnp.float32)]
            + [pltpu.MemorySpace.VMEM((CHUNK, width), jnp.float32)
               for _ in range(nbuf)]
            + [pltpu.SemaphoreType.DMA for _ in range(nbuf)]
        ),
    )


_agg_l1 = _make_agg(128, src_cstride=EPAD // CHUNK, dst_cstride=0,
                    cpt=EPAD // NS // CHUNK, n_passes=2, nbuf=2)
_agg_l2 = _make_agg(DOUT, src_cstride=EPAD // NC // CHUNK,
                    dst_cstride=EPAD // NC // CHUNK,
                    cpt=EPAD // (NC * NS) // CHUNK, n_passes=1, nbuf=4)


def _dinv_block(deg_ref, i):
    d = deg_ref[0, pl.ds(i * BLK, BLK)] + deg_ref[1, pl.ds(i * BLK, BLK)] + 1.0
    return lax.rsqrt(d)


K2BLK = 2048


def _k2_body(x_ref, w_ref, deg_ref, out_ref):
    i = pl.program_id(0)
    d = (deg_ref[0, pl.ds(i * K2BLK, K2BLK)]
         + deg_ref[1, pl.ds(i * K2BLK, K2BLK)] + 1.0)
    dinv = lax.rsqrt(d)
    h = jnp.dot(x_ref[...], w_ref[...], preferred_element_type=jnp.float32)
    out_ref[...] = h * dinv[:, None]


def _k2_call(x, W1, deg):
    return pl.pallas_call(
        _k2_body,
        grid=(NPAD // K2BLK, 2),
        in_specs=[
            pl.BlockSpec((K2BLK, DIN), lambda i, j: (i, 0)),
            pl.BlockSpec((DIN, 128), lambda i, j: (0, j)),
            pl.BlockSpec((2, NPAD), lambda i, j: (0, 0)),
        ],
        out_specs=pl.BlockSpec((K2BLK, 128),
                               lambda i, j: (j * (NPAD // K2BLK) + i, 0)),
        out_shape=jax.ShapeDtypeStruct((NC * NPAD, 128), jnp.float32),
    )(x, W1, deg)


def _k4_body(alo, ahi, glo, ghi, mlo, mhi, deg_ref, b1_ref, w2_ref, out_ref):
    i = pl.program_id(0)
    dinv = _dinv_block(deg_ref, i)[:, None]
    zlo = mlo[0] * (2.0 * jnp.maximum(
        dinv * (alo[0] + glo[0]) + b1_ref[0, pl.ds(0, 128)][None, :], 0.0))
    zhi = mhi[0] * (2.0 * jnp.maximum(
        dinv * (ahi[0] + ghi[0]) + b1_ref[0, pl.ds(128, 128)][None, :], 0.0))
    h2 = (jnp.dot(zlo, w2_ref[pl.ds(0, 128), :], preferred_element_type=jnp.float32)
          + jnp.dot(zhi, w2_ref[pl.ds(128, 128), :], preferred_element_type=jnp.float32))
    out_ref[...] = h2 * dinv


def _k4_call(agg1, g1, mask3, deg, b1, W2):
    half = lambda h: pl.BlockSpec((1, BLK, 128), lambda i, h=h: (h, i, 0))
    return pl.pallas_call(
        _k4_body,
        grid=(NPAD // BLK,),
        in_specs=[
            half(0), half(1),
            half(0), half(1),
            half(0), half(1),
            pl.BlockSpec((2, NPAD), lambda i: (0, 0)),
            pl.BlockSpec((1, DHID), lambda i: (0, 0)),
            pl.BlockSpec((DHID, DOUT), lambda i: (0, 0)),
        ],
        out_specs=pl.BlockSpec((BLK, DOUT), lambda i: (i, 0)),
        out_shape=jax.ShapeDtypeStruct((NPAD, DOUT), jnp.float32),
    )(agg1, agg1, g1, g1, mask3, mask3, deg, b1, W2)


def _k6_body(p0_ref, p1_ref, g2_ref, deg_ref, b2_ref, out_ref):
    i = pl.program_id(0)
    dinv = _dinv_block(deg_ref, i)[:, None]
    out_ref[...] = (dinv * (p0_ref[...] + p1_ref[...] + g2_ref[...])
                    + b2_ref[0][None, :])


def _k6_call(agg2, g2, deg, b2):
    nb = NPAD // BLK
    return pl.pallas_call(
        _k6_body,
        grid=(nb,),
        in_specs=[
            pl.BlockSpec((BLK, DOUT), lambda i: (i, 0)),
            pl.BlockSpec((BLK, DOUT), lambda i: (nb + i, 0)),
            pl.BlockSpec((BLK, DOUT), lambda i: (i, 0)),
            pl.BlockSpec((2, NPAD), lambda i: (0, 0)),
            pl.BlockSpec((1, DOUT), lambda i: (0, 0)),
        ],
        out_specs=pl.BlockSpec((BLK, DOUT), lambda i: (i, 0)),
        out_shape=jax.ShapeDtypeStruct((N, DOUT), jnp.float32),
    )(agg2, agg2, g2, deg, b2)


def kernel(x, edge_index, W1, b1, W2, b2):
    src = edge_index[0].astype(jnp.int32)
    dst = edge_index[1].astype(jnp.int32)
    pad_idx = (jnp.arange(EPAD - E, dtype=jnp.int32) % 192) + (NPAD - 192)
    srcp = jnp.concatenate([src, pad_idx])
    dstp = jnp.concatenate([dst, pad_idx])
    src2 = jnp.concatenate([srcp, srcp + NPAD])

    mask = jax.random.bernoulli(jax.random.key(42), 0.5, (N, DHID))
    mask_pad = jnp.pad(mask.astype(jnp.float32), ((0, NPAD - N), (0, 0)))
    mask3 = jnp.stack([mask_pad[:, :128], mask_pad[:, 128:]])
    z128 = jnp.zeros((RPT, 128), jnp.float32)
    z64 = jnp.zeros((RPT, DOUT), jnp.float32)

    srcp2 = srcp.reshape(-1, CHUNK)
    dstp2 = dstp.reshape(-1, CHUNK)
    src22 = src2.reshape(-1, CHUNK)

    deg = _deg_kernel(dstp2).reshape(2, NPAD)
    g1 = _k2_call(x, W1, deg)
    agg1 = _agg_l1(g1, src22, dstp2, z128)
    g2 = _k4_call(agg1.reshape(2, NPAD, 128), g1.reshape(2, NPAD, 128),
                  mask3, deg, b1.reshape(1, DHID), W2)
    agg2 = _agg_l2(g2, srcp2, dstp2, z64)
    return _k6_call(agg2, g2, deg, b2.reshape(1, DOUT))


# speedup vs baseline: 22.7163x; 1.0077x over previous
import functools

import jax
import jax.numpy as jnp
from jax import lax
from jax.experimental import pallas as pl
from jax.experimental.pallas import tpu as pltpu
from jax.experimental.pallas import tpu_sc as plsc

N = 10000
E = 160000
NPAD = 10240
EPAD = 163840
DIN = 256
DHID = 256
DOUT = 64
NC = 2
NS = 16
CHUNK = 128
RPT = NPAD // NS
BLK = 1024

_MESH = dict(core_axis_name="c", subcore_axis_name="s")


_DEG_CPT = EPAD // (NC * NS) // CHUNK


def _deg_body(dst_hbm, deg_out, didx, ones_v, zrow_v, acc, sem):
    c = lax.axis_index("c")
    s = lax.axis_index("s")

    def fill_ones(i, _):
        ones_v[pl.ds(i * 16, 16)] = jnp.full((16,), 1.0, jnp.float32)
        return 0

    lax.fori_loop(0, CHUNK // 16, fill_ones, 0)

    def fill_zero(i, _):
        zrow_v[pl.ds(i * 16, 16)] = jnp.zeros((16,), jnp.float32)
        return 0

    lax.fori_loop(0, RPT // 16, fill_zero, 0)
    pltpu.sync_copy(
        dst_hbm.at[pl.ds(c * (NC * _DEG_CPT * NS // 2) + s * _DEG_CPT, _DEG_CPT)],
        didx)
    pltpu.sync_copy(zrow_v, acc.at[pl.ds(s * RPT, RPT)])
    plsc.subcore_barrier()

    def issue(g, _):
        pltpu.async_copy(ones_v, acc.at[didx.at[g]], sem, add=True)
        return 0

    lax.fori_loop(0, _DEG_CPT, issue, 0)

    def drain(g, _):
        pltpu.make_async_copy(ones_v, acc.at[didx.at[0]], sem).wait()
        return 0

    lax.fori_loop(0, _DEG_CPT, drain, 0)
    plsc.subcore_barrier()
    pltpu.sync_copy(acc.at[pl.ds(s * RPT, RPT)],
                    deg_out.at[pl.ds(c * NPAD + s * RPT, RPT)])


_deg_kernel = pl.kernel(
    _deg_body,
    out_type=jax.ShapeDtypeStruct((NC * NPAD,), jnp.float32),
    mesh=plsc.VectorSubcoreMesh(**_MESH),
    scratch_types=[
        pltpu.MemorySpace.VMEM((_DEG_CPT, CHUNK), jnp.int32),
        pltpu.MemorySpace.VMEM((CHUNK,), jnp.float32),
        pltpu.MemorySpace.VMEM((RPT,), jnp.float32),
        pltpu.MemorySpace.VMEM_SHARED((NPAD,), jnp.float32),
        pltpu.SemaphoreType.DMA,
    ],
)


def _agg_body(src_cstride, dst_cstride, cpt, n_passes, nbuf,
              g_hbm, src_hbm, dst_hbm, z_hbm, out_hbm,
              sidx, didx, acc, *bufs):
    c = lax.axis_index("c")
    s = lax.axis_index("s")
    ppt = cpt // n_passes
    rows = bufs[:nbuf]
    sems = bufs[nbuf:]
    pltpu.sync_copy(z_hbm, acc.at[pl.ds(s * RPT, RPT)])
    plsc.subcore_barrier()

    for p in range(n_passes):
        sbase = c * src_cstride + s * cpt + p * ppt
        dbase = c * dst_cstride + s * cpt + p * ppt
        pltpu.sync_copy(src_hbm.at[pl.ds(sbase, ppt)], sidx)
        pltpu.sync_copy(dst_hbm.at[pl.ds(dbase, ppt)], didx)
        for b in range(nbuf):
            pltpu.async_copy(g_hbm.at[sidx.at[b]], rows[b], sems[b])

        def macro(m, _):
            for b in range(nbuf):
                ch = m * nbuf + b
                pltpu.make_async_copy(
                    g_hbm.at[sidx.at[ch]], rows[b], sems[b]).wait()
                pltpu.sync_copy(rows[b], acc.at[didx.at[ch]], add=True)
                pltpu.async_copy(g_hbm.at[sidx.at[ch + nbuf]], rows[b], sems[b])
            return 0

        lax.fori_loop(0, ppt // nbuf - 1, macro, 0)
        for b in range(nbuf):
            ch = ppt - nbuf + b
            pltpu.make_async_copy(g_hbm.at[sidx.at[ch]], rows[b], sems[b]).wait()
            pltpu.sync_copy(rows[b], acc.at[didx.at[ch]], add=True)

    plsc.subcore_barrier()
    pltpu.sync_copy(acc.at[pl.ds(s * RPT, RPT)],
                    out_hbm.at[pl.ds(c * NPAD + s * RPT, RPT)])


def _make_agg(width, src_cstride, dst_cstride, cpt, n_passes, nbuf):
    params = pltpu.CompilerParams(use_tc_tiling_on_sc=(width % 128 == 0))
    return pl.kernel(
        functools.partial(_agg_body, src_cstride, dst_cstride, cpt, n_passes,
                          nbuf),
        out_type=jax.ShapeDtypeStruct((NC * NPAD, width), jnp.float32),
        mesh=plsc.VectorSubcoreMesh(**_MESH),
        compiler_params=params,
        scratch_types=(
            [pltpu.MemorySpace.VMEM((cpt // n_passes, CHUNK), jnp.int32),
             pltpu.MemorySpace.VMEM((cpt // n_passes, CHUNK), jnp.int32),
             pltpu.MemorySpace.VMEM_SHARED((NPAD, width), jnp.float32)]
            + [pltpu.MemorySpace.VMEM((CHUNK, width), jnp.float32)
               for _ in range(nbuf)]
            + [pltpu.SemaphoreType.DMA for _ in range(nbuf)]
        ),
    )


_agg_l1 = _make_agg(128, src_cstride=EPAD // CHUNK, dst_cstride=0,
                    cpt=EPAD // NS // CHUNK, n_passes=2, nbuf=2)
_agg_l2 = _make_agg(DOUT, src_cstride=EPAD // NC // CHUNK,
                    dst_cstride=EPAD // NC // CHUNK,
                    cpt=EPAD // (NC * NS) // CHUNK, n_passes=1, nbuf=8)


def _dinv_block(deg_ref, i):
    d = deg_ref[0, pl.ds(i * BLK, BLK)] + deg_ref[1, pl.ds(i * BLK, BLK)] + 1.0
    return lax.rsqrt(d)


K2BLK = 2048


def _k2_body(x_ref, w_ref, deg_ref, out_ref):
    i = pl.program_id(0)
    d = (deg_ref[0, pl.ds(i * K2BLK, K2BLK)]
         + deg_ref[1, pl.ds(i * K2BLK, K2BLK)] + 1.0)
    dinv = lax.rsqrt(d)
    h = jnp.dot(x_ref[...], w_ref[...], preferred_element_type=jnp.float32)
    out_ref[...] = h * dinv[:, None]


def _k2_call(x, W1, deg):
    return pl.pallas_call(
        _k2_body,
        grid=(NPAD // K2BLK, 2),
        in_specs=[
            pl.BlockSpec((K2BLK, DIN), lambda i, j: (i, 0)),
            pl.BlockSpec((DIN, 128), lambda i, j: (0, j)),
            pl.BlockSpec((2, NPAD), lambda i, j: (0, 0)),
        ],
        out_specs=pl.BlockSpec((K2BLK, 128),
                               lambda i, j: (j * (NPAD // K2BLK) + i, 0)),
        out_shape=jax.ShapeDtypeStruct((NC * NPAD, 128), jnp.float32),
    )(x, W1, deg)


def _k4_body(alo, ahi, glo, ghi, mlo, mhi, deg_ref, b1_ref, w2_ref, out_ref):
    i = pl.program_id(0)
    dinv = _dinv_block(deg_ref, i)[:, None]
    zlo = jnp.where(mlo[0] != 0, 2.0 * jnp.maximum(
        dinv * (alo[0] + glo[0]) + b1_ref[0, pl.ds(0, 128)][None, :], 0.0), 0.0)
    zhi = jnp.where(mhi[0] != 0, 2.0 * jnp.maximum(
        dinv * (ahi[0] + ghi[0]) + b1_ref[0, pl.ds(128, 128)][None, :], 0.0), 0.0)
    h2 = (jnp.dot(zlo, w2_ref[pl.ds(0, 128), :], preferred_element_type=jnp.float32)
          + jnp.dot(zhi, w2_ref[pl.ds(128, 128), :], preferred_element_type=jnp.float32))
    out_ref[...] = h2 * dinv


def _k4_call(agg1, g1, mask3, deg, b1, W2):
    half = lambda h: pl.BlockSpec((1, BLK, 128), lambda i, h=h: (h, i, 0))
    return pl.pallas_call(
        _k4_body,
        grid=(NPAD // BLK,),
        in_specs=[
            half(0), half(1),
            half(0), half(1),
            half(0), half(1),
            pl.BlockSpec((2, NPAD), lambda i: (0, 0)),
            pl.BlockSpec((1, DHID), lambda i: (0, 0)),
            pl.BlockSpec((DHID, DOUT), lambda i: (0, 0)),
        ],
        out_specs=pl.BlockSpec((BLK, DOUT), lambda i: (i, 0)),
        out_shape=jax.ShapeDtypeStruct((NPAD, DOUT), jnp.float32),
    )(agg1, agg1, g1, g1, mask3, mask3, deg, b1, W2)


def _k6_body(p0_ref, p1_ref, g2_ref, deg_ref, b2_ref, out_ref):
    i = pl.program_id(0)
    dinv = _dinv_block(deg_ref, i)[:, None]
    out_ref[...] = (dinv * (p0_ref[...] + p1_ref[...] + g2_ref[...])
                    + b2_ref[0][None, :])


def _k6_call(agg2, g2, deg, b2):
    nb = NPAD // BLK
    return pl.pallas_call(
        _k6_body,
        grid=(nb,),
        in_specs=[
            pl.BlockSpec((BLK, DOUT), lambda i: (i, 0)),
            pl.BlockSpec((BLK, DOUT), lambda i: (nb + i, 0)),
            pl.BlockSpec((BLK, DOUT), lambda i: (i, 0)),
            pl.BlockSpec((2, NPAD), lambda i: (0, 0)),
            pl.BlockSpec((1, DOUT), lambda i: (0, 0)),
        ],
        out_specs=pl.BlockSpec((BLK, DOUT), lambda i: (i, 0)),
        out_shape=jax.ShapeDtypeStruct((N, DOUT), jnp.float32),
    )(agg2, agg2, g2, deg, b2)


def kernel(x, edge_index, W1, b1, W2, b2):
    src = edge_index[0].astype(jnp.int32)
    dst = edge_index[1].astype(jnp.int32)
    pad_idx = (jnp.arange(EPAD - E, dtype=jnp.int32) % 192) + (NPAD - 192)
    srcp = jnp.concatenate([src, pad_idx])
    dstp = jnp.concatenate([dst, pad_idx])
    src2 = jnp.concatenate([srcp, srcp + NPAD])

    mask = jax.random.bernoulli(jax.random.key(42), 0.5, (N, DHID))
    mask_pad = jnp.pad(mask.astype(jnp.int8), ((0, NPAD - N), (0, 0)))
    mask3 = jnp.stack([mask_pad[:, :128], mask_pad[:, 128:]])
    z128 = jnp.zeros((RPT, 128), jnp.float32)
    z64 = jnp.zeros((RPT, DOUT), jnp.float32)

    dstp2 = dstp.reshape(-1, CHUNK)
    src22 = src2.reshape(-1, CHUNK)

    deg = _deg_kernel(dstp2).reshape(2, NPAD)
    g1 = _k2_call(x, W1, deg)
    agg1 = _agg_l1(g1, src22, dstp2, z128)
    g2 = _k4_call(agg1.reshape(2, NPAD, 128), g1.reshape(2, NPAD, 128),
                  mask3, deg, b1.reshape(1, DHID), W2)
    agg2 = _agg_l2(g2, src22, dstp2, z64)
    return _k6_call(agg2, g2, deg, b2.reshape(1, DOUT))
